# baseline (speedup 1.0000x reference)
import functools

import jax
import jax.numpy as jnp
from jax import lax
from jax.experimental import pallas as pl
from jax.experimental.pallas import tpu as pltpu
from jax.experimental.pallas import tpu_sc as plsc

_NC = 2
_NS = 16
_CH = 80
_RD = 4


@functools.lru_cache(maxsize=None)
def _make_sc_agg(N, NF, EW, D, with_deg):
  ch, rd = 80, 4
  ri = 2 * rd
  NCH = EW // ch
  tail = EW - NCH * ch
  rt = N // _NS
  assert N % (_NS * 8) == 0 and EW % 8 == 0 and tail % 8 == 0

  mesh = plsc.VectorSubcoreMesh(
      core_axis_name="c", subcore_axis_name="s",
      num_cores=_NC, num_subcores=_NS)

  out_type = [jax.ShapeDtypeStruct((_NC * N, D), jnp.float32)]
  scratch = [
      [pltpu.VMEM((ch,), jnp.int32) for _ in range(ri)],
      [pltpu.VMEM((ch,), jnp.int32) for _ in range(ri)],
      [pltpu.VMEM((ch, D), jnp.float32) for _ in range(rd)],
      [pltpu.SemaphoreType.DMA for _ in range(ri)],
      [pltpu.SemaphoreType.DMA for _ in range(rd)],
      [pltpu.SemaphoreType.DMA for _ in range(rd)],
      pltpu.VMEM_SHARED((N, D), jnp.float32),
      pltpu.VMEM((ch,), jnp.float32),
      [pltpu.VMEM((max(tail, 8),), jnp.int32) for _ in range(2)],
  ]
  if with_deg:
    out_type.append(jax.ShapeDtypeStruct((_NC * N,), jnp.float32))
    scratch.append(pltpu.VMEM_SHARED((N,), jnp.float32))
    scratch.append(pltpu.VMEM((ch,), jnp.float32))

  @functools.partial(
      pl.kernel,
      mesh=mesh,
      compiler_params=pltpu.CompilerParams(use_tc_tiling_on_sc=False),
      out_type=out_type,
      scratch_types=scratch,
  )
  def sc_agg(feat_hbm, src_hbm, dst_hbm, *refs):
    if with_deg:
      (out_hbm, deg_hbm, sbuf, dbuf, rows, isems, gsems, ssems, acc, zero_v,
       tbuf, dacc, ones_v) = refs
    else:
      out_hbm, sbuf, dbuf, rows, isems, gsems, ssems, acc, zero_v, tbuf = refs

    c = lax.axis_index("c")
    s = lax.axis_index("s")
    r0 = s * rt
    def zloop(j, carry):
      rows[0][j // (D // 16), pl.ds((j % (D // 16)) * 16, 16)] = (
          jnp.zeros((16,), jnp.float32))
      return carry
    lax.fori_loop(0, ch * D // 16, zloop, 0)
    for i in range(ch // 16):
      zero_v[pl.ds(i * 16, 16)] = jnp.zeros((16,), jnp.float32)
      if with_deg:
        ones_v[pl.ds(i * 16, 16)] = jnp.ones((16,), jnp.float32)
    nz = rt // ch
    for j in range(nz):
      pltpu.sync_copy(rows[0].at[pl.ds(0, ch)],
                      acc.at[pl.ds(r0 + j * ch, ch)])
      if with_deg:
        pltpu.sync_copy(zero_v, dacc.at[pl.ds(r0 + j * ch, ch)])
    rem = rt - nz * ch
    if rem:
      pltpu.sync_copy(rows[0].at[pl.ds(0, rem)],
                      acc.at[pl.ds(r0 + nz * ch, rem)])
      if with_deg:
        pltpu.sync_copy(zero_v.at[pl.ds(0, rem)],
                        dacc.at[pl.ds(r0 + nz * ch, rem)])
    plsc.subcore_barrier()

    wid = c * _NS + s

    def idx_copies(k, bi):
      return (pltpu.make_async_copy(src_hbm.at[wid, pl.ds(k * ch, ch)],
                                    sbuf[bi], isems[bi]),
              pltpu.make_async_copy(dst_hbm.at[wid, pl.ds(k * ch, ch)],
                                    dbuf[bi], isems[bi]))

    def gather_copy(br, bi):
      return pltpu.make_async_copy(feat_hbm.at[sbuf[bi]], rows[br], gsems[br])

    def scatter_start(br, bi):
      if with_deg:
        pltpu.async_copy(ones_v, dacc.at[dbuf[bi]], ssems[br], add=True)
      pltpu.async_copy(rows[br], acc.at[dbuf[bi]], ssems[br], add=True)

    def scatter_wait(br, bi):
      if with_deg:
        pltpu.make_async_copy(ones_v, dacc.at[dbuf[bi]], ssems[br]).wait()
      pltpu.make_async_copy(rows[br], acc.at[dbuf[bi]], ssems[br]).wait()

    for j in range(ri - 2):
      for cp in idx_copies(j, j):
        cp.start()
    for j in range(2):
      for cp in idx_copies(j, j):
        cp.wait()
      gather_copy(j, j).start()

    def turn(k, j):
      k = jnp.int32(k)
      @pl.when(k >= 2)
      def _():
        scatter_wait((j - 2) % rd, (j - 2) % ri)

      @pl.when(k + ri - 2 < NCH)
      def _():
        for cp in idx_copies(k + ri - 2, (j + ri - 2) % ri):
          cp.start()

      @pl.when(k + 2 < NCH)
      def _():
        for cp in idx_copies(k + 2, (j + 2) % ri):
          cp.wait()
        gather_copy((j + 2) % rd, (j + 2) % ri).start()

      gather_copy(j % rd, j % ri).wait()
      scatter_start(j % rd, j % ri)

    def body(o, carry):
      for j in range(ri):
        turn(ri * o + j, j)
      return carry

    lax.fori_loop(0, NCH // ri, body, 0)
    for k in range(NCH - NCH % ri, NCH):
      turn(k, k % ri)
    for k in (NCH - 2, NCH - 1):
      scatter_wait(k % rd, k % ri)
    if tail:
      pltpu.sync_copy(src_hbm.at[wid, pl.ds(NCH * ch, tail)], tbuf[0])
      pltpu.sync_copy(dst_hbm.at[wid, pl.ds(NCH * ch, tail)], tbuf[1])
      pltpu.async_copy(feat_hbm.at[tbuf[0]], rows[0].at[pl.ds(0, tail)],
                       gsems[0]).wait()
      if with_deg:
        pltpu.sync_copy(ones_v.at[pl.ds(0, tail)], dacc.at[tbuf[1]], add=True)
      pltpu.sync_copy(rows[0].at[pl.ds(0, tail)], acc.at[tbuf[1]], add=True)
    plsc.subcore_barrier()
    pltpu.sync_copy(acc.at[pl.ds(r0, rt)],
                    out_hbm.at[pl.ds(c * N + r0, rt)])
    if with_deg:
      pltpu.sync_copy(dacc.at[pl.ds(r0, rt)],
                      deg_hbm.at[pl.ds(c * N + r0, rt)])

  return sc_agg


def _tc_layer1(x, p3, deg3, w_self, w_neigh, b):
  n, d = x.shape
  bn = 1000
  assert n % bn == 0

  def body(x_ref, p0_ref, p1_ref, d0_ref, d1_ref, ws_ref, wn_ref, b_ref,
           h_ref):
    dg = d0_ref[0] + d1_ref[0]
    dinv = 1.0 / jnp.maximum(dg, 1.0)
    agg = (p0_ref[0] + p1_ref[0]) * dinv
    h = (jnp.dot(x_ref[...], ws_ref[...], preferred_element_type=jnp.float32)
         + jnp.dot(agg, wn_ref[...], preferred_element_type=jnp.float32)
         + b_ref[...])
    h_ref[...] = jnp.maximum(h, 0.0)

  return pl.pallas_call(
      body,
      grid=(n // bn,),
      in_specs=[
          pl.BlockSpec((bn, d), lambda i: (i, 0)),
          pl.BlockSpec((1, bn, d), lambda i: (0, i, 0)),
          pl.BlockSpec((1, bn, d), lambda i: (1, i, 0)),
          pl.BlockSpec((1, bn, 1), lambda i: (0, i, 0)),
          pl.BlockSpec((1, bn, 1), lambda i: (1, i, 0)),
          pl.BlockSpec((d, d), lambda i: (0, 0)),
          pl.BlockSpec((d, d), lambda i: (0, 0)),
          pl.BlockSpec((1, d), lambda i: (0, 0)),
      ],
      out_specs=pl.BlockSpec((bn, d), lambda i: (i, 0)),
      out_shape=jax.ShapeDtypeStruct((n, d), jnp.float32),
  )(x, p3, p3, deg3, deg3, w_self, w_neigh, b.reshape(1, d))


def _tc_layer2(h, q3, deg3, w_self, w_neigh, b, w_fc, b_fc):
  n, d = h.shape
  co = w_fc.shape[1]
  bn = 1000
  assert n % bn == 0

  def body(h_ref, q0_ref, q1_ref, d0_ref, d1_ref, ws_ref, wn_ref, b_ref,
           wfc_ref, bfc_ref, logits_ref, h2_ref):
    dg = d0_ref[0] + d1_ref[0]
    dinv = 1.0 / jnp.maximum(dg, 1.0)
    agg = (q0_ref[0] + q1_ref[0]) * dinv
    h2 = (jnp.dot(h_ref[...], ws_ref[...], preferred_element_type=jnp.float32)
          + jnp.dot(agg, wn_ref[...], preferred_element_type=jnp.float32)
          + b_ref[...])
    h2_ref[...] = h2
    logits_ref[...] = (
        jnp.dot(h2, wfc_ref[...], preferred_element_type=jnp.float32)
        + bfc_ref[...])

  return pl.pallas_call(
      body,
      grid=(n // bn,),
      in_specs=[
          pl.BlockSpec((bn, d), lambda i: (i, 0)),
          pl.BlockSpec((1, bn, d), lambda i: (0, i, 0)),
          pl.BlockSpec((1, bn, d), lambda i: (1, i, 0)),
          pl.BlockSpec((1, bn, 1), lambda i: (0, i, 0)),
          pl.BlockSpec((1, bn, 1), lambda i: (1, i, 0)),
          pl.BlockSpec((d, d), lambda i: (0, 0)),
          pl.BlockSpec((d, d), lambda i: (0, 0)),
          pl.BlockSpec((1, d), lambda i: (0, 0)),
          pl.BlockSpec((d, co), lambda i: (0, 0)),
          pl.BlockSpec((1, co), lambda i: (0, 0)),
      ],
      out_specs=[pl.BlockSpec((bn, co), lambda i: (i, 0)),
                 pl.BlockSpec((bn, d), lambda i: (i, 0))],
      out_shape=[jax.ShapeDtypeStruct((n, co), jnp.float32),
                 jax.ShapeDtypeStruct((n, d), jnp.float32)],
  )(h, q3, q3, deg3, deg3, w_self, w_neigh, b.reshape(1, d), w_fc,
    b_fc.reshape(1, co))


def kernel(x, edge_index, W_self1, W_neigh1, b1, W_self2, W_neigh2, b2,
           W_fc, b_fc):
  n, d = x.shape
  e = edge_index.shape[1]
  nwk = _NC * _NS
  n2 = ((n + _NS * 8 - 1) // (_NS * 8)) * (_NS * 8)

  assert e % (nwk * 8) == 0
  ew = e // nwk
  src = edge_index[0].astype(jnp.int32).reshape(nwk, ew)
  dst = edge_index[1].astype(jnp.int32).reshape(nwk, ew)

  p, deg = _make_sc_agg(n2, n, ew, d, True)(x, src, dst)
  p3 = p.reshape(_NC, n2, d)
  deg3 = deg.reshape(_NC, n2, 1)
  h = _tc_layer1(x, p3, deg3, W_self1, W_neigh1, b1)
  (q,) = _make_sc_agg(n2, n, ew, d, False)(h, src, dst)
  q3 = q.reshape(_NC, n2, d)
  logits, h2 = _tc_layer2(h, q3, deg3, W_self2, W_neigh2, b2, W_fc, b_fc)
  return (logits, h2)
